# baseline (speedup 1.0000x reference)
import functools

import jax
import jax.numpy as jnp
from jax import lax
from jax.experimental import pallas as pl
from jax.experimental.pallas import tpu as pltpu
from jax.experimental.pallas import tpu_sc as plsc

N = 10000
NP = 10240
E = 320000
D = 128
HID = 256
NCLS = 64

NTILES = 16
NSC = 2
CHUNK = 80
CHUNKS_PER_TILE = E // (NSC * NTILES * CHUNK)
SCHUNK = 48
SSTAGES = 6
SHALF = 36
NBUF = 6
EPAD = NSC * NTILES * SSTAGES * SHALF * SCHUNK
ROWS_PER_TILE = NP // NTILES

_mesh = plsc.VectorSubcoreMesh(core_axis_name="c", subcore_axis_name="s")


@functools.partial(
    pl.kernel,
    out_type=jax.ShapeDtypeStruct((NSC * NP,), jnp.float32),
    mesh=_mesh,
    scratch_types=[
        pltpu.VMEM((CHUNKS_PER_TILE, CHUNK), jnp.int32),
        pltpu.VMEM((CHUNK,), jnp.float32),
        pltpu.VMEM_SHARED((NP,), jnp.float32),
    ],
)
def _deg_kernel(dst_hbm, z1_hbm, out_hbm, dst_v, ones_v, deg_sh):
    c = lax.axis_index("c")
    s = lax.axis_index("s")
    w = c * NTILES + s

    @pl.when(s == 0)
    def _():
        pltpu.sync_copy(z1_hbm, deg_sh)

    for j in range(CHUNK // 16):
        ones_v[pl.ds(j * 16, 16)] = jnp.ones((16,), jnp.float32)
    pltpu.sync_copy(dst_hbm.at[w], dst_v)
    plsc.subcore_barrier()

    def body(j, carry):
        pltpu.sync_copy(ones_v, deg_sh.at[dst_v.at[j]], add=True)
        return carry

    lax.fori_loop(0, CHUNKS_PER_TILE, body, 0)
    plsc.subcore_barrier()

    @pl.when(s == 0)
    def _():
        pltpu.sync_copy(deg_sh, out_hbm.at[pl.ds(c * NP, NP)])


@functools.partial(
    pl.kernel,
    out_type=jax.ShapeDtypeStruct((NSC * NP, D), jnp.float32),
    mesh=_mesh,
    scratch_types=[
        pltpu.VMEM((SHALF, SCHUNK), jnp.int32),
        pltpu.VMEM((SHALF, SCHUNK), jnp.int32),
        pltpu.VMEM((SCHUNK, D), jnp.float32),
        pltpu.VMEM((SCHUNK, D), jnp.float32),
        pltpu.VMEM((SCHUNK, D), jnp.float32),
        pltpu.VMEM((SCHUNK, D), jnp.float32),
        pltpu.VMEM((SCHUNK, D), jnp.float32),
        pltpu.VMEM((SCHUNK, D), jnp.float32),
        pltpu.SemaphoreType.DMA,
        pltpu.SemaphoreType.DMA,
        pltpu.SemaphoreType.DMA,
        pltpu.SemaphoreType.DMA,
        pltpu.SemaphoreType.DMA,
        pltpu.SemaphoreType.DMA,
        pltpu.SemaphoreType.DMA,
        pltpu.SemaphoreType.DMA,
        pltpu.SemaphoreType.DMA,
        pltpu.SemaphoreType.DMA,
        pltpu.SemaphoreType.DMA,
        pltpu.SemaphoreType.DMA,
        pltpu.VMEM_SHARED((NP, D), jnp.float32),
    ],
)
def _spmv_kernel(xp_hbm, src_hbm, dst_hbm, znd_hbm, out_hbm,
                 src_v, dst_v, rows0, rows1, rows2, rows3, rows4, rows5,
                 g0, g1, g2, g3, g4, g5, s0, s1, s2, s3, s4, s5, acc_sh):
    c = lax.axis_index("c")
    s = lax.axis_index("s")
    w = c * NTILES + s
    rows = (rows0, rows1, rows2, rows3, rows4, rows5)
    gsem = (g0, g1, g2, g3, g4, g5)
    ssem = (s0, s1, s2, s3, s4, s5)

    pltpu.sync_copy(znd_hbm.at[pl.ds(s * ROWS_PER_TILE, ROWS_PER_TILE)],
                    acc_sh.at[pl.ds(s * ROWS_PER_TILE, ROWS_PER_TILE)])
    plsc.subcore_barrier()

    def gather(j, b):
        pltpu.async_copy(xp_hbm.at[src_v.at[j]], rows[b], gsem[b])

    def gather_wait(j, b):
        pltpu.make_async_copy(xp_hbm.at[src_v.at[j]], rows[b], gsem[b]).wait()

    def scat(j, b):
        pltpu.async_copy(rows[b], acc_sh.at[dst_v.at[j]], ssem[b], add=True)

    def scat_wait(j, b):
        pltpu.make_async_copy(rows[b], acc_sh.at[dst_v.at[j]], ssem[b]).wait()

    for h in range(SSTAGES):
        pltpu.sync_copy(src_hbm.at[w, h], src_v)
        pltpu.sync_copy(dst_hbm.at[w, h], dst_v)

        for b in range(NBUF):
            gather(b, b)

        def body(j4, carry):
            base = NBUF * j4
            for b in range(NBUF):
                gather_wait(base + b, b)
                scat(base + b, b)
            for b in range(NBUF):
                scat_wait(base + b, b)
                gather(base + NBUF + b, b)
            return carry

        lax.fori_loop(0, SHALF // NBUF - 1, body, 0)

        base = SHALF - NBUF
        for b in range(NBUF):
            gather_wait(base + b, b)
            scat(base + b, b)
        for b in range(NBUF):
            scat_wait(base + b, b)

    plsc.subcore_barrier()
    pltpu.sync_copy(acc_sh.at[pl.ds(s * ROWS_PER_TILE, ROWS_PER_TILE)],
                    out_hbm.at[pl.ds(c * NP + s * ROWS_PER_TILE, ROWS_PER_TILE)])


def _norm_body(degp_ref, feats_ref, nmat_ref, xp0_ref):
    d = degp_ref[0] + degp_ref[1]
    nrm = lax.rsqrt(jnp.maximum(d, 1.0))
    nmat = jnp.broadcast_to(nrm, (NP, D))
    nmat_ref[...] = nmat
    xp0_ref[...] = nmat * feats_ref[...]


_norm_call = pl.pallas_call(
    _norm_body,
    out_shape=[jax.ShapeDtypeStruct((NP, D), jnp.float32)] * 2,
)


def _scale_body(s0_ref, s1_ref, nm_ref, yin_ref, yout_ref, xp_ref):
    total = s0_ref[...] + s1_ref[...]
    x = nm_ref[...] * total
    yout_ref[...] = yin_ref[...] + x
    xp_ref[...] = nm_ref[...] * x


_BLK = 512
_scale_call = pl.pallas_call(
    _scale_body,
    grid=(NP // _BLK,),
    in_specs=[pl.BlockSpec((_BLK, D), lambda i: (i, 0))] * 4,
    out_specs=[pl.BlockSpec((_BLK, D), lambda i: (i, 0))] * 2,
    out_shape=[jax.ShapeDtypeStruct((NP, D), jnp.float32)] * 2,
)


def _mlp_body(s0_ref, s1_ref, nm_ref, yin_ref, w1_ref, b1_ref, w2_ref, b2_ref,
              out_ref):
    y = yin_ref[...] + nm_ref[...] * (s0_ref[...] + s1_ref[...])
    h = y * 0.25
    h = jnp.dot(h, w1_ref[...], preferred_element_type=jnp.float32) + b1_ref[...]
    h = jnp.maximum(h, 0.0)
    z = jnp.dot(h, w2_ref[...], preferred_element_type=jnp.float32) + b2_ref[...]
    m = jnp.max(z, axis=-1, keepdims=True)
    lse = jnp.log(jnp.sum(jnp.exp(z - m), axis=-1, keepdims=True)) + m
    out_ref[...] = z - lse


_mlp_call = pl.pallas_call(
    _mlp_body,
    grid=(NP // _BLK,),
    in_specs=[
        pl.BlockSpec((_BLK, D), lambda i: (i, 0)),
        pl.BlockSpec((_BLK, D), lambda i: (i, 0)),
        pl.BlockSpec((_BLK, D), lambda i: (i, 0)),
        pl.BlockSpec((_BLK, D), lambda i: (i, 0)),
        pl.BlockSpec((D, HID), lambda i: (0, 0)),
        pl.BlockSpec((1, HID), lambda i: (0, 0)),
        pl.BlockSpec((HID, NCLS), lambda i: (0, 0)),
        pl.BlockSpec((1, NCLS), lambda i: (0, 0)),
    ],
    out_specs=pl.BlockSpec((_BLK, NCLS), lambda i: (i, 0)),
    out_shape=jax.ShapeDtypeStruct((NP, NCLS), jnp.float32),
)


def kernel(feats, edge_index, W1, b1, W2, b2):
    pad = N + jnp.arange(EPAD - E, dtype=jnp.int32) % (NP - N)
    src = jnp.concatenate([edge_index[0], pad]).reshape(
        NSC * NTILES, SSTAGES, SHALF, SCHUNK)
    dst = jnp.concatenate([edge_index[1], pad]).reshape(
        NSC * NTILES, SSTAGES, SHALF, SCHUNK)
    dst_deg = edge_index[1].reshape(NSC * NTILES, CHUNKS_PER_TILE, CHUNK)
    feats_p = jnp.pad(feats, ((0, NP - N), (0, 0)))
    z1 = jnp.zeros((NP,), jnp.float32)
    znd = jnp.zeros((NP, D), jnp.float32)

    degp = _deg_kernel(dst_deg, z1)
    nmat, xp = _norm_call(degp.reshape(NSC, NP, 1), feats_p)

    y = feats_p
    for _ in range(2):
        sp = _spmv_kernel(xp, src, dst, znd)
        y, xp = _scale_call(sp[:NP], sp[NP:], nmat, y)
    sp = _spmv_kernel(xp, src, dst, znd)

    out = _mlp_call(sp[:NP], sp[NP:], nmat, y,
                    W1, b1.reshape(1, HID), W2, b2.reshape(1, NCLS))
    return out[:N]


# speedup vs baseline: 1.0177x; 1.0177x over previous
import functools

import jax
import jax.numpy as jnp
from jax import lax
from jax.experimental import pallas as pl
from jax.experimental.pallas import tpu as pltpu
from jax.experimental.pallas import tpu_sc as plsc

N = 10000
NP = 10240
E = 320000
D = 128
HID = 256
NCLS = 64

NTILES = 16
NSC = 2
CHUNK = 80
CHUNKS_PER_TILE = E // (NSC * NTILES * CHUNK)
SCHUNK = 72
SSTAGES = 4
SHALF = 36
NBUF = 4
EPAD = NSC * NTILES * SSTAGES * SHALF * SCHUNK
ROWS_PER_TILE = NP // NTILES

_mesh = plsc.VectorSubcoreMesh(core_axis_name="c", subcore_axis_name="s")


@functools.partial(
    pl.kernel,
    out_type=jax.ShapeDtypeStruct((NSC * NP,), jnp.float32),
    mesh=_mesh,
    scratch_types=[
        pltpu.VMEM((CHUNKS_PER_TILE, CHUNK), jnp.int32),
        pltpu.VMEM((CHUNK,), jnp.float32),
        pltpu.VMEM_SHARED((NP,), jnp.float32),
    ],
)
def _deg_kernel(dst_hbm, z1_hbm, out_hbm, dst_v, ones_v, deg_sh):
    c = lax.axis_index("c")
    s = lax.axis_index("s")
    w = c * NTILES + s

    @pl.when(s == 0)
    def _():
        pltpu.sync_copy(z1_hbm, deg_sh)

    for j in range(CHUNK // 16):
        ones_v[pl.ds(j * 16, 16)] = jnp.ones((16,), jnp.float32)
    pltpu.sync_copy(dst_hbm.at[w], dst_v)
    plsc.subcore_barrier()

    def body(j, carry):
        pltpu.sync_copy(ones_v, deg_sh.at[dst_v.at[j]], add=True)
        return carry

    lax.fori_loop(0, CHUNKS_PER_TILE, body, 0)
    plsc.subcore_barrier()

    @pl.when(s == 0)
    def _():
        pltpu.sync_copy(deg_sh, out_hbm.at[pl.ds(c * NP, NP)])


@functools.partial(
    pl.kernel,
    out_type=jax.ShapeDtypeStruct((NSC * NP, D), jnp.float32),
    mesh=_mesh,
    scratch_types=[
        pltpu.VMEM((SHALF, SCHUNK), jnp.int32),
        pltpu.VMEM((SHALF, SCHUNK), jnp.int32),
        pltpu.VMEM((SCHUNK, D), jnp.float32),
        pltpu.VMEM((SCHUNK, D), jnp.float32),
        pltpu.VMEM((SCHUNK, D), jnp.float32),
        pltpu.VMEM((SCHUNK, D), jnp.float32),
        pltpu.SemaphoreType.DMA,
        pltpu.SemaphoreType.DMA,
        pltpu.SemaphoreType.DMA,
        pltpu.SemaphoreType.DMA,
        pltpu.SemaphoreType.DMA,
        pltpu.SemaphoreType.DMA,
        pltpu.SemaphoreType.DMA,
        pltpu.SemaphoreType.DMA,
        pltpu.VMEM_SHARED((NP, D), jnp.float32),
    ],
)
def _spmv_kernel(xp_hbm, src_hbm, dst_hbm, znd_hbm, out_hbm,
                 src_v, dst_v, rows0, rows1, rows2, rows3,
                 g0, g1, g2, g3, s0, s1, s2, s3, acc_sh):
    c = lax.axis_index("c")
    s = lax.axis_index("s")
    w = c * NTILES + s
    rows = (rows0, rows1, rows2, rows3)
    gsem = (g0, g1, g2, g3)
    ssem = (s0, s1, s2, s3)

    pltpu.sync_copy(znd_hbm.at[pl.ds(s * ROWS_PER_TILE, ROWS_PER_TILE)],
                    acc_sh.at[pl.ds(s * ROWS_PER_TILE, ROWS_PER_TILE)])
    plsc.subcore_barrier()

    def gather(j, b):
        pltpu.async_copy(xp_hbm.at[src_v.at[j]], rows[b], gsem[b])

    def gather_wait(j, b):
        pltpu.make_async_copy(xp_hbm.at[src_v.at[j]], rows[b], gsem[b]).wait()

    def scat(j, b):
        pltpu.async_copy(rows[b], acc_sh.at[dst_v.at[j]], ssem[b], add=True)

    def scat_wait(j, b):
        pltpu.make_async_copy(rows[b], acc_sh.at[dst_v.at[j]], ssem[b]).wait()

    for h in range(SSTAGES):
        pltpu.sync_copy(src_hbm.at[w, h], src_v)
        pltpu.sync_copy(dst_hbm.at[w, h], dst_v)

        for b in range(NBUF):
            gather(b, b)

        def body(j4, carry):
            base = NBUF * j4
            for b in range(NBUF):
                gather_wait(base + b, b)
                scat(base + b, b)
            for b in range(NBUF):
                scat_wait(base + b, b)
                gather(base + NBUF + b, b)
            return carry

        lax.fori_loop(0, SHALF // NBUF - 1, body, 0)

        base = SHALF - NBUF
        for b in range(NBUF):
            gather_wait(base + b, b)
            scat(base + b, b)
        for b in range(NBUF):
            scat_wait(base + b, b)

    plsc.subcore_barrier()
    pltpu.sync_copy(acc_sh.at[pl.ds(s * ROWS_PER_TILE, ROWS_PER_TILE)],
                    out_hbm.at[pl.ds(c * NP + s * ROWS_PER_TILE, ROWS_PER_TILE)])


def _norm_body(degp_ref, feats_ref, nmat_ref, xp0_ref):
    d = degp_ref[0] + degp_ref[1]
    nrm = lax.rsqrt(jnp.maximum(d, 1.0))
    nmat = jnp.broadcast_to(nrm, (NP, D))
    nmat_ref[...] = nmat
    xp0_ref[...] = nmat * feats_ref[...]


_norm_call = pl.pallas_call(
    _norm_body,
    out_shape=[jax.ShapeDtypeStruct((NP, D), jnp.float32)] * 2,
)


def _scale_body(s0_ref, s1_ref, nm_ref, yin_ref, yout_ref, xp_ref):
    total = s0_ref[...] + s1_ref[...]
    x = nm_ref[...] * total
    yout_ref[...] = yin_ref[...] + x
    xp_ref[...] = nm_ref[...] * x


_BLK = 512
_scale_call = pl.pallas_call(
    _scale_body,
    grid=(NP // _BLK,),
    in_specs=[pl.BlockSpec((_BLK, D), lambda i: (i, 0))] * 4,
    out_specs=[pl.BlockSpec((_BLK, D), lambda i: (i, 0))] * 2,
    out_shape=[jax.ShapeDtypeStruct((NP, D), jnp.float32)] * 2,
)


def _mlp_body(s0_ref, s1_ref, nm_ref, yin_ref, w1_ref, b1_ref, w2_ref, b2_ref,
              out_ref):
    y = yin_ref[...] + nm_ref[...] * (s0_ref[...] + s1_ref[...])
    h = y * 0.25
    h = jnp.dot(h, w1_ref[...], preferred_element_type=jnp.float32) + b1_ref[...]
    h = jnp.maximum(h, 0.0)
    z = jnp.dot(h, w2_ref[...], preferred_element_type=jnp.float32) + b2_ref[...]
    m = jnp.max(z, axis=-1, keepdims=True)
    lse = jnp.log(jnp.sum(jnp.exp(z - m), axis=-1, keepdims=True)) + m
    out_ref[...] = z - lse


_mlp_call = pl.pallas_call(
    _mlp_body,
    grid=(NP // _BLK,),
    in_specs=[
        pl.BlockSpec((_BLK, D), lambda i: (i, 0)),
        pl.BlockSpec((_BLK, D), lambda i: (i, 0)),
        pl.BlockSpec((_BLK, D), lambda i: (i, 0)),
        pl.BlockSpec((_BLK, D), lambda i: (i, 0)),
        pl.BlockSpec((D, HID), lambda i: (0, 0)),
        pl.BlockSpec((1, HID), lambda i: (0, 0)),
        pl.BlockSpec((HID, NCLS), lambda i: (0, 0)),
        pl.BlockSpec((1, NCLS), lambda i: (0, 0)),
    ],
    out_specs=pl.BlockSpec((_BLK, NCLS), lambda i: (i, 0)),
    out_shape=jax.ShapeDtypeStruct((NP, NCLS), jnp.float32),
)


def kernel(feats, edge_index, W1, b1, W2, b2):
    pad = N + jnp.arange(EPAD - E, dtype=jnp.int32) % (NP - N)
    src = jnp.concatenate([edge_index[0], pad]).reshape(
        NSC * NTILES, SSTAGES, SHALF, SCHUNK)
    dst = jnp.concatenate([edge_index[1], pad]).reshape(
        NSC * NTILES, SSTAGES, SHALF, SCHUNK)
    dst_deg = edge_index[1].reshape(NSC * NTILES, CHUNKS_PER_TILE, CHUNK)
    feats_p = jnp.pad(feats, ((0, NP - N), (0, 0)))
    z1 = jnp.zeros((NP,), jnp.float32)
    znd = jnp.zeros((NP, D), jnp.float32)

    degp = _deg_kernel(dst_deg, z1)
    nmat, xp = _norm_call(degp.reshape(NSC, NP, 1), feats_p)

    y = feats_p
    for _ in range(2):
        sp = _spmv_kernel(xp, src, dst, znd)
        y, xp = _scale_call(sp[:NP], sp[NP:], nmat, y)
    sp = _spmv_kernel(xp, src, dst, znd)

    out = _mlp_call(sp[:NP], sp[NP:], nmat, y,
                    W1, b1.reshape(1, HID), W2, b2.reshape(1, NCLS))
    return out[:N]
